# Initial kernel scaffold; baseline (speedup 1.0000x reference)
import functools

import jax
import jax.numpy as jnp
from jax import lax
from jax.experimental import pallas as pl
from jax.experimental.pallas import tpu as pltpu
from jax.experimental.pallas import tpu_sc as plsc

N = 10000
NPAD = 10240
E = 320000
D_IN, D_H, D_OUT = 128, 128, 64

NC, NS = 2, 16
NW = NC * NS
EPT = E // NW
CH = 80
NCHUNK = EPT // CH
RPT = NPAD // NS

_mesh = plsc.VectorSubcoreMesh(core_axis_name="c", subcore_axis_name="s")


def _deg_body(dst_hbm, w_hbm, deg0_hbm, deg1_hbm,
              degv, idxb, wb, accb, tmp, sdeg):
    cid = lax.axis_index("c")
    sid = lax.axis_index("s")
    wid = cid * NS + sid

    def zero(i, _):
        degv[pl.ds(i * 16, 16)] = jnp.zeros((16,), jnp.float32)
        return 0
    lax.fori_loop(0, NPAD // 16, zero, 0)

    ebase = wid * EPT

    def chunk(k, _):
        off = ebase + k * CH
        pltpu.sync_copy(dst_hbm.at[pl.ds(off, CH)], idxb)
        pltpu.sync_copy(w_hbm.at[pl.ds(off, CH)], wb)

        def inner(i, _):
            idx16 = idxb[pl.ds(i * 16, 16)]
            w16 = wb[pl.ds(i * 16, 16)]
            plsc.addupdate_scatter(degv, [idx16], w16)
            return 0
        lax.fori_loop(0, CH // 16, inner, 0)
        return 0
    lax.fori_loop(0, NCHUNK, chunk, 0)

    pltpu.sync_copy(degv, sdeg.at[sid])
    plsc.subcore_barrier()
    rbase = sid * RPT
    pltpu.sync_copy(sdeg.at[0, pl.ds(rbase, RPT)], accb)

    def red(t, _):
        pltpu.sync_copy(sdeg.at[t, pl.ds(rbase, RPT)], tmp)

        def addv(i, _):
            sl = pl.ds(i * 16, 16)
            accb[sl] = accb[sl] + tmp[sl]
            return 0
        lax.fori_loop(0, RPT // 16, addv, 0)
        return 0
    lax.fori_loop(1, NS, red, 0)

    @pl.when(cid == 0)
    def _():
        pltpu.sync_copy(accb, deg0_hbm.at[pl.ds(rbase, RPT)])

    @pl.when(cid == 1)
    def _():
        pltpu.sync_copy(accb, deg1_hbm.at[pl.ds(rbase, RPT)])


_deg_call = functools.partial(
    pl.kernel,
    out_type=(jax.ShapeDtypeStruct((NPAD,), jnp.float32),
              jax.ShapeDtypeStruct((NPAD,), jnp.float32)),
    mesh=_mesh,
    scratch_types=[
        pltpu.VMEM((NPAD,), jnp.float32),
        pltpu.VMEM((CH,), jnp.int32),
        pltpu.VMEM((CH,), jnp.float32),
        pltpu.VMEM((RPT,), jnp.float32),
        pltpu.VMEM((RPT,), jnp.float32),
        pltpu.VMEM_SHARED((NS, NPAD), jnp.float32),
    ],
)(_deg_body)


def _agg_body(y_hbm, src_hbm, dst_hbm, w_hbm, z_hbm, out_hbm,
              acc, idxs, idxd, wb, rows, sem, *, d):
    cid = lax.axis_index("c")
    sid = lax.axis_index("s")
    wid = cid * NS + sid
    rbase = sid * RPT

    pltpu.sync_copy(z_hbm, acc.at[pl.ds(rbase, RPT)])
    plsc.subcore_barrier()

    ebase = wid * EPT

    def chunk(k, _):
        off = ebase + k * CH
        pltpu.sync_copy(src_hbm.at[pl.ds(off, CH)], idxs)
        pltpu.sync_copy(dst_hbm.at[pl.ds(off, CH)], idxd)
        pltpu.sync_copy(w_hbm.at[pl.ds(off, CH)], wb)
        pltpu.async_copy(y_hbm.at[idxs], rows, sem).wait()

        def scale(i, _):
            wi = wb[i]
            for j in range(d // 16):
                sl = pl.ds(j * 16, 16)
                rows[i, sl] = rows[i, sl] * wi
            return 0
        lax.fori_loop(0, CH, scale, 0)

        pltpu.sync_copy(rows, acc.at[idxd], add=True)
        return 0
    lax.fori_loop(0, NCHUNK, chunk, 0)

    plsc.subcore_barrier()
    pltpu.sync_copy(acc.at[pl.ds(rbase, RPT)],
                    out_hbm.at[cid, pl.ds(rbase, RPT)])


def _make_agg(d):
    return pl.kernel(
        functools.partial(_agg_body, d=d),
        out_type=jax.ShapeDtypeStruct((NC, NPAD, d), jnp.float32),
        mesh=_mesh,
        scratch_types=[
            pltpu.VMEM_SHARED((NPAD, d), jnp.float32),
            pltpu.VMEM((CH,), jnp.int32),
            pltpu.VMEM((CH,), jnp.int32),
            pltpu.VMEM((CH,), jnp.float32),
            pltpu.VMEM((CH, d), jnp.float32),
            pltpu.SemaphoreType.DMA,
        ],
    )


_agg_h = _make_agg(D_H)
_agg_o = _make_agg(D_OUT)


BM = 1024
GRID = NPAD // BM


def _tc1_body(x_ref, w_ref, d0_ref, d1_ref, y_ref, dinv_ref):
    deg = d0_ref[...] + d1_ref[...] + 1.0
    dinv = lax.rsqrt(deg)
    xw = jnp.dot(x_ref[...], w_ref[...], preferred_element_type=jnp.float32)
    y_ref[...] = xw * dinv
    dinv_ref[...] = dinv


def _tc1(x, W1, d0, d1):
    return pl.pallas_call(
        _tc1_body,
        grid=(GRID,),
        in_specs=[
            pl.BlockSpec((BM, D_IN), lambda i: (i, 0)),
            pl.BlockSpec((D_IN, D_H), lambda i: (0, 0)),
            pl.BlockSpec((BM, 1), lambda i: (i, 0)),
            pl.BlockSpec((BM, 1), lambda i: (i, 0)),
        ],
        out_specs=[
            pl.BlockSpec((BM, D_H), lambda i: (i, 0)),
            pl.BlockSpec((BM, 1), lambda i: (i, 0)),
        ],
        out_shape=[
            jax.ShapeDtypeStruct((N, D_H), jnp.float32),
            jax.ShapeDtypeStruct((NPAD, 1), jnp.float32),
        ],
    )(x, W1, d0, d1)


def _tc2_body(p_ref, y1_ref, dinv_ref, b1_ref, w2_ref, y2_ref):
    agg = p_ref[0] + p_ref[1] + y1_ref[...]
    h = jnp.maximum(dinv_ref[...] * agg + b1_ref[...], 0.0)
    y2_ref[...] = jnp.dot(h, w2_ref[...],
                          preferred_element_type=jnp.float32) * dinv_ref[...]


def _tc2(p1, y1, dinv, b1, W2):
    return pl.pallas_call(
        _tc2_body,
        grid=(GRID,),
        in_specs=[
            pl.BlockSpec((NC, BM, D_H), lambda i: (0, i, 0)),
            pl.BlockSpec((BM, D_H), lambda i: (i, 0)),
            pl.BlockSpec((BM, 1), lambda i: (i, 0)),
            pl.BlockSpec((1, D_H), lambda i: (0, 0)),
            pl.BlockSpec((D_H, D_OUT), lambda i: (0, 0)),
        ],
        out_specs=pl.BlockSpec((BM, D_OUT), lambda i: (i, 0)),
        out_shape=jax.ShapeDtypeStruct((N, D_OUT), jnp.float32),
    )(p1, y1, dinv, b1, W2)


def _tc3_body(p_ref, y2_ref, dinv_ref, b2_ref, o_ref):
    agg = p_ref[0] + p_ref[1] + y2_ref[...]
    o = dinv_ref[...] * agg + b2_ref[...]
    m = jnp.max(o, axis=-1, keepdims=True)
    e = jnp.exp(o - m)
    lse = jnp.log(jnp.sum(e, axis=-1, keepdims=True))
    o_ref[...] = (o - m) - lse


def _tc3(p2, y2, dinv, b2):
    return pl.pallas_call(
        _tc3_body,
        grid=(GRID,),
        in_specs=[
            pl.BlockSpec((NC, BM, D_OUT), lambda i: (0, i, 0)),
            pl.BlockSpec((BM, D_OUT), lambda i: (i, 0)),
            pl.BlockSpec((BM, 1), lambda i: (i, 0)),
            pl.BlockSpec((1, D_OUT), lambda i: (0, 0)),
        ],
        out_specs=pl.BlockSpec((BM, D_OUT), lambda i: (i, 0)),
        out_shape=jax.ShapeDtypeStruct((N, D_OUT), jnp.float32),
    )(p2, y2, dinv, b2)


def kernel(x, edge_index, edge_weight, W1, b1, W2, b2):
    src = edge_index[0].astype(jnp.int32)
    dst = edge_index[1].astype(jnp.int32)
    ew = edge_weight.astype(jnp.float32)

    deg0, deg1 = _deg_call(dst, ew)
    d0 = deg0.reshape(NPAD, 1)
    d1 = deg1.reshape(NPAD, 1)

    y1, dinv = _tc1(x, W1, d0, d1)

    z_h = jnp.zeros((RPT, D_H), jnp.float32)
    p1 = _agg_h(y1, src, dst, ew, z_h)

    y2 = _tc2(p1, y1, dinv, b1.reshape(1, D_H), W2)

    z_o = jnp.zeros((RPT, D_OUT), jnp.float32)
    p2 = _agg_o(y2, src, dst, ew, z_o)

    return _tc3(p2, y2, dinv, b2.reshape(1, D_OUT))


# speedup vs baseline: 10.4036x; 10.4036x over previous
import functools

import jax
import jax.numpy as jnp
from jax import lax
from jax.experimental import pallas as pl
from jax.experimental.pallas import tpu as pltpu
from jax.experimental.pallas import tpu_sc as plsc

N = 10000
NPAD = 10240
E = 320000
D_IN, D_H, D_OUT = 128, 128, 64

NC, NS = 2, 16
NW = NC * NS
EPT = E // NW
CH = 80
NCHUNK = EPT // CH
RPT = NPAD // NS

_mesh = plsc.VectorSubcoreMesh(core_axis_name="c", subcore_axis_name="s")
_sc_params = pltpu.CompilerParams(needs_layout_passes=False,
                                  use_tc_tiling_on_sc=False)


def _deg_body(dst_hbm, w_hbm, deg0_hbm, deg1_hbm,
              degv, idxb, wb, accb, tmp, sdeg):
    cid = lax.axis_index("c")
    sid = lax.axis_index("s")
    wid = cid * NS + sid

    def zero(i, _):
        degv[pl.ds(i * 16, 16)] = jnp.zeros((16,), jnp.float32)
        return 0
    lax.fori_loop(0, NPAD // 16, zero, 0)

    ebase = wid * EPT

    def chunk(k, _):
        off = ebase + k * CH
        pltpu.sync_copy(dst_hbm.at[pl.ds(off, CH)], idxb)
        pltpu.sync_copy(w_hbm.at[pl.ds(off, CH)], wb)

        def inner(i, _):
            idx16 = idxb[pl.ds(i * 16, 16)]
            w16 = wb[pl.ds(i * 16, 16)]
            plsc.addupdate_scatter(degv, [idx16], w16)
            return 0
        lax.fori_loop(0, CH // 16, inner, 0)
        return 0
    lax.fori_loop(0, NCHUNK, chunk, 0)

    pltpu.sync_copy(degv, sdeg.at[sid])
    plsc.subcore_barrier()
    rbase = sid * RPT
    pltpu.sync_copy(sdeg.at[0, pl.ds(rbase, RPT)], accb)

    def red(t, _):
        pltpu.sync_copy(sdeg.at[t, pl.ds(rbase, RPT)], tmp)

        def addv(i, _):
            sl = pl.ds(i * 16, 16)
            accb[sl] = accb[sl] + tmp[sl]
            return 0
        lax.fori_loop(0, RPT // 16, addv, 0)
        return 0
    lax.fori_loop(1, NS, red, 0)

    @pl.when(cid == 0)
    def _():
        pltpu.sync_copy(accb, deg0_hbm.at[pl.ds(rbase, RPT)])

    @pl.when(cid == 1)
    def _():
        pltpu.sync_copy(accb, deg1_hbm.at[pl.ds(rbase, RPT)])


_deg_call = functools.partial(
    pl.kernel,
    out_type=(jax.ShapeDtypeStruct((NPAD,), jnp.float32),
              jax.ShapeDtypeStruct((NPAD,), jnp.float32)),
    mesh=_mesh,
    scratch_types=[
        pltpu.VMEM((NPAD,), jnp.float32),
        pltpu.VMEM((CH,), jnp.int32),
        pltpu.VMEM((CH,), jnp.float32),
        pltpu.VMEM((RPT,), jnp.float32),
        pltpu.VMEM((RPT,), jnp.float32),
        pltpu.VMEM_SHARED((NS, NPAD), jnp.float32),
    ],
    compiler_params=_sc_params,
)(_deg_body)


def _agg_body(y_hbm, src_hbm, dst_hbm, w_hbm, z_hbm, out_hbm,
              acc, idxs, idxd, wb, rows, sem, *, d):
    cid = lax.axis_index("c")
    sid = lax.axis_index("s")
    wid = cid * NS + sid
    rbase = sid * RPT

    pltpu.sync_copy(z_hbm, acc.at[pl.ds(rbase, RPT)])
    plsc.subcore_barrier()

    ebase = wid * EPT

    def chunk(k, _):
        off = ebase + k * CH
        pltpu.sync_copy(src_hbm.at[pl.ds(off, CH)], idxs)
        pltpu.sync_copy(dst_hbm.at[pl.ds(off, CH)], idxd)
        pltpu.sync_copy(w_hbm.at[pl.ds(off, CH)], wb)
        pltpu.async_copy(y_hbm.at[idxs], rows, sem).wait()

        def scale(g, _):
            w16 = wb[pl.ds(g * 16, 16)]
            for e in range(16):
                we = w16[e]
                r = g * 16 + e
                for j in range(d // 16):
                    sl = pl.ds(j * 16, 16)
                    rows[r, sl] = rows[r, sl] * we
            return 0
        lax.fori_loop(0, CH // 16, scale, 0)

        pltpu.sync_copy(rows, acc.at[idxd], add=True)
        return 0
    lax.fori_loop(0, NCHUNK, chunk, 0)

    plsc.subcore_barrier()
    pltpu.sync_copy(acc.at[pl.ds(rbase, RPT)],
                    out_hbm.at[cid, pl.ds(rbase, RPT)])


def _make_agg(d):
    return pl.kernel(
        functools.partial(_agg_body, d=d),
        out_type=jax.ShapeDtypeStruct((NC, NPAD, d), jnp.float32),
        mesh=_mesh,
        scratch_types=[
            pltpu.VMEM_SHARED((NPAD, d), jnp.float32),
            pltpu.VMEM((CH,), jnp.int32),
            pltpu.VMEM((CH,), jnp.int32),
            pltpu.VMEM((CH,), jnp.float32),
            pltpu.VMEM((CH, d), jnp.float32),
            pltpu.SemaphoreType.DMA,
        ],
        compiler_params=_sc_params,
    )


_agg_h = _make_agg(D_H)
_agg_o = _make_agg(D_OUT)


BM = 1024
GRID = NPAD // BM


def _tc1_body(x_ref, w_ref, d0_ref, d1_ref, y_ref, dinv_ref):
    deg = d0_ref[...] + d1_ref[...] + 1.0
    dinv = lax.rsqrt(deg)
    xw = jnp.dot(x_ref[...], w_ref[...], preferred_element_type=jnp.float32)
    y_ref[...] = xw * dinv
    dinv_ref[...] = dinv


def _tc1(x, W1, d0, d1):
    return pl.pallas_call(
        _tc1_body,
        grid=(GRID,),
        in_specs=[
            pl.BlockSpec((BM, D_IN), lambda i: (i, 0)),
            pl.BlockSpec((D_IN, D_H), lambda i: (0, 0)),
            pl.BlockSpec((BM, 1), lambda i: (i, 0)),
            pl.BlockSpec((BM, 1), lambda i: (i, 0)),
        ],
        out_specs=[
            pl.BlockSpec((BM, D_H), lambda i: (i, 0)),
            pl.BlockSpec((BM, 1), lambda i: (i, 0)),
        ],
        out_shape=[
            jax.ShapeDtypeStruct((N, D_H), jnp.float32),
            jax.ShapeDtypeStruct((NPAD, 1), jnp.float32),
        ],
    )(x, W1, d0, d1)


def _tc2_body(p_ref, y1_ref, dinv_ref, b1_ref, w2_ref, y2_ref):
    agg = p_ref[0] + p_ref[1] + y1_ref[...]
    h = jnp.maximum(dinv_ref[...] * agg + b1_ref[...], 0.0)
    y2_ref[...] = jnp.dot(h, w2_ref[...],
                          preferred_element_type=jnp.float32) * dinv_ref[...]


def _tc2(p1, y1, dinv, b1, W2):
    return pl.pallas_call(
        _tc2_body,
        grid=(GRID,),
        in_specs=[
            pl.BlockSpec((NC, BM, D_H), lambda i: (0, i, 0)),
            pl.BlockSpec((BM, D_H), lambda i: (i, 0)),
            pl.BlockSpec((BM, 1), lambda i: (i, 0)),
            pl.BlockSpec((1, D_H), lambda i: (0, 0)),
            pl.BlockSpec((D_H, D_OUT), lambda i: (0, 0)),
        ],
        out_specs=pl.BlockSpec((BM, D_OUT), lambda i: (i, 0)),
        out_shape=jax.ShapeDtypeStruct((N, D_OUT), jnp.float32),
    )(p1, y1, dinv, b1, W2)


def _tc3_body(p_ref, y2_ref, dinv_ref, b2_ref, o_ref):
    agg = p_ref[0] + p_ref[1] + y2_ref[...]
    o = dinv_ref[...] * agg + b2_ref[...]
    m = jnp.max(o, axis=-1, keepdims=True)
    e = jnp.exp(o - m)
    lse = jnp.log(jnp.sum(e, axis=-1, keepdims=True))
    o_ref[...] = (o - m) - lse


def _tc3(p2, y2, dinv, b2):
    return pl.pallas_call(
        _tc3_body,
        grid=(GRID,),
        in_specs=[
            pl.BlockSpec((NC, BM, D_OUT), lambda i: (0, i, 0)),
            pl.BlockSpec((BM, D_OUT), lambda i: (i, 0)),
            pl.BlockSpec((BM, 1), lambda i: (i, 0)),
            pl.BlockSpec((1, D_OUT), lambda i: (0, 0)),
        ],
        out_specs=pl.BlockSpec((BM, D_OUT), lambda i: (i, 0)),
        out_shape=jax.ShapeDtypeStruct((N, D_OUT), jnp.float32),
    )(p2, y2, dinv, b2)


def kernel(x, edge_index, edge_weight, W1, b1, W2, b2):
    src = edge_index[0].astype(jnp.int32)
    dst = edge_index[1].astype(jnp.int32)
    ew = edge_weight.astype(jnp.float32)

    deg0, deg1 = _deg_call(dst, ew)
    d0 = deg0.reshape(NPAD, 1)
    d1 = deg1.reshape(NPAD, 1)

    y1, dinv = _tc1(x, W1, d0, d1)

    z_h = jnp.zeros((RPT, D_H), jnp.float32)
    p1 = _agg_h(y1, src, dst, ew, z_h)

    y2 = _tc2(p1, y1, dinv, b1.reshape(1, D_H), W2)

    z_o = jnp.zeros((RPT, D_OUT), jnp.float32)
    p2 = _agg_o(y2, src, dst, ew, z_o)

    return _tc3(p2, y2, dinv, b2.reshape(1, D_OUT))
